# baseline (speedup 1.0000x reference)
import jax
import jax.numpy as jnp
import numpy as np
from jax import lax
from jax.experimental import pallas as pl
from jax.experimental.pallas import tpu as pltpu
from jax.experimental.pallas import tpu_sc as plsc

C = 96
CSC = 64
CTC = C - CSC
H = 512
W = 512
FH = 128
FW = 128
L = FH * FW
NW = 32
PR_PER_W = FH // NW
ROWS_PER_W = 4 * PR_PER_W
LW = PR_PER_W * FW
CG = 4
NB = 2 * CG

with jax.default_device(jax.devices("cpu")[0]):
    _SIDX = np.asarray(
        jax.random.randint(jax.random.key(42), (L,), 0, 16, jnp.int32))

_S2 = _SIDX.reshape(FH, FW)
_J = np.arange(W)
_U = np.stack(
    [(_S2[:, _J >> 2] == kr * 4 + (_J & 3)) for kr in range(4)]
).astype(np.float32)


def _slab_src(x_hbm, wid, c):
    return x_hbm.at[c, pl.ds(wid * ROWS_PER_W, ROWS_PER_W), :]


def _sc_body(x_hbm, s_hbm, out_hbm, sbuf, ibuf, xbufs, obuf, load_sem,
             store_sem, sem_s):
    cid = lax.axis_index("c")
    sid = lax.axis_index("s")
    wid = sid * 2 + cid
    base_l = wid * LW

    pltpu.async_copy(s_hbm.at[pl.ds(base_l, LW)], sbuf, sem_s)
    for b in range(NB):
        pltpu.async_copy(_slab_src(x_hbm, wid, b), xbufs[b], load_sem)

    pltpu.make_async_copy(s_hbm.at[pl.ds(base_l, LW)], sbuf, sem_s).wait()
    lane4 = lax.iota(jnp.int32, 16) * 4

    @plsc.parallel_loop(0, LW // 16, unroll=4)
    def _prolog(i):
        s = sbuf[pl.ds(i * 16, 16)]
        row = (s >> 2) + (i >> 3) * 4
        col = (s & 3) + lane4 + (i & 7) * 64
        ibuf[pl.ds(i * 16, 16)] = (row << 9) | col

    def group(g, carry):
        for half in range(NB // CG):
            c0 = g * NB + half * CG
            for k in range(CG):
                pltpu.make_async_copy(
                    _slab_src(x_hbm, wid, c0 + k), xbufs[half * CG + k],
                    load_sem,
                ).wait()

            @plsc.parallel_loop(0, LW // 16, unroll=4)
            def _sweep(i):
                p = ibuf[pl.ds(i * 16, 16)]
                row = p >> 9
                col = p & (W - 1)
                for k in range(CG):
                    obuf[c0 + k, pl.ds(i * 16, 16)] = plsc.load_gather(
                        xbufs[half * CG + k], [row, col]
                    )

            @pl.when(c0 + NB + CG <= CSC)
            def _():
                for k in range(CG):
                    pltpu.async_copy(
                        _slab_src(x_hbm, wid, c0 + NB + k),
                        xbufs[half * CG + k], load_sem,
                    )

            pltpu.async_copy(
                obuf.at[pl.ds(c0, CG), :],
                out_hbm.at[pl.ds(c0, CG), pl.ds(base_l, LW)],
                store_sem,
            )

        return carry

    lax.fori_loop(0, CSC // NB, group, 0)
    for g in range(CSC // CG):
        pltpu.make_async_copy(
            obuf.at[pl.ds(g * CG, CG), :],
            out_hbm.at[pl.ds(g * CG, CG), pl.ds(base_l, LW)],
            store_sem,
        ).wait()


def _sc_body_wrap(x_hbm, s_hbm, out_hbm, sbuf, ibuf, *rest):
    xbufs = rest[:NB]
    obuf, load_sem, store_sem, sem_s = rest[NB:]
    _sc_body(x_hbm, s_hbm, out_hbm, sbuf, ibuf, xbufs, obuf, load_sem,
             store_sem, sem_s)


def _tc_body(x_ref, u_ref, out_ref):
    xq = x_ref[0].reshape(FH, 4, W)
    z = xq[:, 0, :] * u_ref[0]
    for kr in range(1, 4):
        z += xq[:, kr, :] * u_ref[kr]
    out_ref[0] = jnp.sum(z.reshape(FH, FW, 4), axis=-1)


@jax.jit
def _run(xr, sidx, u):
    mesh = plsc.VectorSubcoreMesh(core_axis_name="c", subcore_axis_name="s")
    sc_fn = pl.kernel(
        _sc_body_wrap,
        out_type=jax.ShapeDtypeStruct((CSC, L), jnp.float32),
        mesh=mesh,
        scratch_types=[
            pltpu.VMEM((LW,), jnp.int32),
            pltpu.VMEM((LW,), jnp.int32),
        ] + [
            pltpu.VMEM((ROWS_PER_W, W), jnp.float32)
            for _ in range(NB)
        ] + [
            pltpu.VMEM((CSC, LW), jnp.float32),
            pltpu.SemaphoreType.DMA,
            pltpu.SemaphoreType.DMA,
            pltpu.SemaphoreType.DMA,
        ],
        compiler_params=pltpu.CompilerParams(needs_layout_passes=False),
    )
    sc_out = sc_fn(xr, sidx)

    tc_out = pl.pallas_call(
        _tc_body,
        grid=(CTC,),
        in_specs=[
            pl.BlockSpec((1, H, W), lambda c: (CSC + c, 0, 0)),
            pl.BlockSpec((4, FH, W), lambda c: (0, 0, 0)),
        ],
        out_specs=pl.BlockSpec((1, FH, FW), lambda c: (c, 0, 0)),
        out_shape=jax.ShapeDtypeStruct((CTC, FH, FW), jnp.float32),
    )(xr, u)

    return sc_out, tc_out


def kernel(x):
    b, c, h, w = x.shape
    sidx = jnp.asarray(_SIDX)
    u = jnp.asarray(_U)
    xr = x.reshape(C, H, W)
    sc_out, tc_out = _run(xr, sidx, u)
    out = jnp.concatenate([sc_out, tc_out.reshape(CTC, L)], axis=0)
    return out.reshape(1, C, L)


# speedup vs baseline: 5.5284x; 5.5284x over previous
import jax
import jax.numpy as jnp
import numpy as np
from jax import lax
from jax.experimental import pallas as pl
from jax.experimental.pallas import tpu as pltpu
from jax.experimental.pallas import tpu_sc as plsc

C = 96
CSC = 64
CTC = C - CSC
H = 512
W = 512
FH = 128
FW = 128
L = FH * FW
NW = 32
PR_PER_W = FH // NW
ROWS_PER_W = 4 * PR_PER_W
LW = PR_PER_W * FW
CG = 4
NB = 2 * CG

with jax.default_device(jax.devices("cpu")[0]):
    _SIDX = np.asarray(
        jax.random.randint(jax.random.key(42), (L,), 0, 16, jnp.int32))

_S2 = _SIDX.reshape(FH, FW)
_J = np.arange(W)
_U = np.stack(
    [(_S2[:, _J >> 2] == kr * 4 + (_J & 3)) for kr in range(4)]
).astype(np.float32)
_R = ((_J[:, None] >> 2) == np.arange(FW)[None, :]).astype(np.float32)


def _slab_src(x_hbm, wid, c):
    return x_hbm.at[c, pl.ds(wid * ROWS_PER_W, ROWS_PER_W), :]


def _sc_body(x_hbm, s_hbm, out_hbm, sbuf, ibuf, xbufs, obuf, load_sem,
             store_sem, sem_s):
    cid = lax.axis_index("c")
    sid = lax.axis_index("s")
    wid = sid * 2 + cid
    base_l = wid * LW

    pltpu.async_copy(s_hbm.at[pl.ds(base_l, LW)], sbuf, sem_s)
    for b in range(NB):
        pltpu.async_copy(_slab_src(x_hbm, wid, b), xbufs[b], load_sem)

    pltpu.make_async_copy(s_hbm.at[pl.ds(base_l, LW)], sbuf, sem_s).wait()
    lane4 = lax.iota(jnp.int32, 16) * 4

    @plsc.parallel_loop(0, LW // 16, unroll=4)
    def _prolog(i):
        s = sbuf[pl.ds(i * 16, 16)]
        row = (s >> 2) + (i >> 3) * 4
        col = (s & 3) + lane4 + (i & 7) * 64
        ibuf[pl.ds(i * 16, 16)] = (row << 9) | col

    def group(g, carry):
        for half in range(NB // CG):
            c0 = g * NB + half * CG
            for k in range(CG):
                pltpu.make_async_copy(
                    _slab_src(x_hbm, wid, c0 + k), xbufs[half * CG + k],
                    load_sem,
                ).wait()

            @plsc.parallel_loop(0, LW // 16, unroll=4)
            def _sweep(i):
                p = ibuf[pl.ds(i * 16, 16)]
                row = p >> 9
                col = p & (W - 1)
                for k in range(CG):
                    obuf[c0 + k, pl.ds(i * 16, 16)] = plsc.load_gather(
                        xbufs[half * CG + k], [row, col]
                    )

            @pl.when(c0 + NB + CG <= CSC)
            def _():
                for k in range(CG):
                    pltpu.async_copy(
                        _slab_src(x_hbm, wid, c0 + NB + k),
                        xbufs[half * CG + k], load_sem,
                    )

            pltpu.async_copy(
                obuf.at[pl.ds(c0, CG), :],
                out_hbm.at[pl.ds(c0, CG), pl.ds(base_l, LW)],
                store_sem,
            )

        return carry

    lax.fori_loop(0, CSC // NB, group, 0)
    for g in range(CSC // CG):
        pltpu.make_async_copy(
            obuf.at[pl.ds(g * CG, CG), :],
            out_hbm.at[pl.ds(g * CG, CG), pl.ds(base_l, LW)],
            store_sem,
        ).wait()


def _sc_body_wrap(x_hbm, s_hbm, out_hbm, sbuf, ibuf, *rest):
    xbufs = rest[:NB]
    obuf, load_sem, store_sem, sem_s = rest[NB:]
    _sc_body(x_hbm, s_hbm, out_hbm, sbuf, ibuf, xbufs, obuf, load_sem,
             store_sem, sem_s)


def _tc_body(x_ref, u_ref, r_ref, out_ref):
    xq = x_ref[0].reshape(FH, 4, W)
    z = xq[:, 0, :] * u_ref[0]
    for kr in range(1, 4):
        z += xq[:, kr, :] * u_ref[kr]
    out_ref[0] = lax.dot_general(
        z, r_ref[...], (((1,), (0,)), ((), ())),
        precision=lax.Precision.HIGHEST,
        preferred_element_type=jnp.float32,
    )


@jax.jit
def _run(xr, sidx, u, r):
    mesh = plsc.VectorSubcoreMesh(core_axis_name="c", subcore_axis_name="s")
    sc_fn = pl.kernel(
        _sc_body_wrap,
        out_type=jax.ShapeDtypeStruct((CSC, L), jnp.float32),
        mesh=mesh,
        scratch_types=[
            pltpu.VMEM((LW,), jnp.int32),
            pltpu.VMEM((LW,), jnp.int32),
        ] + [
            pltpu.VMEM((ROWS_PER_W, W), jnp.float32)
            for _ in range(NB)
        ] + [
            pltpu.VMEM((CSC, LW), jnp.float32),
            pltpu.SemaphoreType.DMA,
            pltpu.SemaphoreType.DMA,
            pltpu.SemaphoreType.DMA,
        ],
        compiler_params=pltpu.CompilerParams(needs_layout_passes=False),
    )
    sc_out = sc_fn(xr, sidx)

    tc_out = pl.pallas_call(
        _tc_body,
        grid=(CTC,),
        in_specs=[
            pl.BlockSpec((1, H, W), lambda c: (CSC + c, 0, 0)),
            pl.BlockSpec((4, FH, W), lambda c: (0, 0, 0)),
            pl.BlockSpec((W, FW), lambda c: (0, 0)),
        ],
        out_specs=pl.BlockSpec((1, FH, FW), lambda c: (c, 0, 0)),
        out_shape=jax.ShapeDtypeStruct((CTC, FH, FW), jnp.float32),
    )(xr, u, r)

    return sc_out, tc_out


def kernel(x):
    b, c, h, w = x.shape
    sidx = jnp.asarray(_SIDX)
    u = jnp.asarray(_U)
    xr = x.reshape(C, H, W)
    sc_out, tc_out = _run(xr, sidx, u, jnp.asarray(_R))
    out = jnp.concatenate([sc_out, tc_out.reshape(CTC, L)], axis=0)
    return out.reshape(1, C, L)


# speedup vs baseline: 5.5309x; 1.0005x over previous
import jax
import jax.numpy as jnp
import numpy as np
from jax import lax
from jax.experimental import pallas as pl
from jax.experimental.pallas import tpu as pltpu
from jax.experimental.pallas import tpu_sc as plsc

C = 96
CSC = 64
CTC = C - CSC
H = 512
W = 512
FH = 128
FW = 128
L = FH * FW
NW = 32
PR_PER_W = FH // NW
ROWS_PER_W = 4 * PR_PER_W
LW = PR_PER_W * FW
CG = 4
NB = 2 * CG

with jax.default_device(jax.devices("cpu")[0]):
    _SIDX = np.asarray(
        jax.random.randint(jax.random.key(42), (L,), 0, 16, jnp.int32))

_S2 = _SIDX.reshape(FH, FW)
_J = np.arange(W)
_U = np.stack(
    [(_S2[:, _J >> 2] == kr * 4 + (_J & 3)) for kr in range(4)]
).astype(np.float32)
_R = ((_J[:, None] >> 2) == np.arange(FW)[None, :]).astype(np.float32)


def _slab_src(x_hbm, wid, c):
    return x_hbm.at[c, pl.ds(wid * ROWS_PER_W, ROWS_PER_W), :]


def _sc_body(x_hbm, s_hbm, out_hbm, sbuf, ibuf, xbufs, obuf, load_sem,
             store_sem, sem_s):
    cid = lax.axis_index("c")
    sid = lax.axis_index("s")
    wid = sid * 2 + cid
    base_l = wid * LW

    pltpu.async_copy(s_hbm.at[pl.ds(base_l, LW)], sbuf, sem_s)
    for b in range(NB):
        pltpu.async_copy(_slab_src(x_hbm, wid, b), xbufs[b], load_sem)

    pltpu.make_async_copy(s_hbm.at[pl.ds(base_l, LW)], sbuf, sem_s).wait()
    lane4 = lax.iota(jnp.int32, 16) * 4

    @plsc.parallel_loop(0, LW // 16, unroll=4)
    def _prolog(i):
        s = sbuf[pl.ds(i * 16, 16)]
        row = (s >> 2) + (i >> 3) * 4
        col = (s & 3) + lane4 + (i & 7) * 64
        ibuf[pl.ds(i * 16, 16)] = (row << 9) | col

    def group(g, carry):
        for half in range(NB // CG):
            c0 = g * NB + half * CG
            for k in range(CG):
                pltpu.make_async_copy(
                    _slab_src(x_hbm, wid, c0 + k), xbufs[half * CG + k],
                    load_sem,
                ).wait()

            @plsc.parallel_loop(0, LW // 16, unroll=4)
            def _sweep(i):
                p = ibuf[pl.ds(i * 16, 16)]
                row = p >> 9
                col = p & (W - 1)
                for k in range(CG):
                    obuf[c0 + k, pl.ds(i * 16, 16)] = plsc.load_gather(
                        xbufs[half * CG + k], [row, col]
                    )

            @pl.when(c0 + NB + CG <= CSC)
            def _():
                for k in range(CG):
                    pltpu.async_copy(
                        _slab_src(x_hbm, wid, c0 + NB + k),
                        xbufs[half * CG + k], load_sem,
                    )

            pltpu.async_copy(
                obuf.at[pl.ds(c0, CG), :],
                out_hbm.at[pl.ds(c0, CG), pl.ds(base_l, LW)],
                store_sem,
            )

        return carry

    lax.fori_loop(0, CSC // NB, group, 0)
    for g in range(CSC // CG):
        pltpu.make_async_copy(
            obuf.at[pl.ds(g * CG, CG), :],
            out_hbm.at[pl.ds(g * CG, CG), pl.ds(base_l, LW)],
            store_sem,
        ).wait()


def _sc_body_wrap(x_hbm, s_hbm, out_hbm, sbuf, ibuf, *rest):
    xbufs = rest[:NB]
    obuf, load_sem, store_sem, sem_s = rest[NB:]
    _sc_body(x_hbm, s_hbm, out_hbm, sbuf, ibuf, xbufs, obuf, load_sem,
             store_sem, sem_s)


def _tc_body(x_ref, u_ref, r_ref, out_ref):
    xq = x_ref[0].reshape(FH, 4, W)
    z = xq[:, 0, :] * u_ref[0]
    for kr in range(1, 4):
        z += xq[:, kr, :] * u_ref[kr]
    out_ref[0] = lax.dot_general(
        z, r_ref[...], (((1,), (0,)), ((), ())),
        precision=lax.Precision.HIGHEST,
        preferred_element_type=jnp.float32,
    )


@jax.jit
def _run(xr, sidx, u, r):
    mesh = plsc.VectorSubcoreMesh(core_axis_name="c", subcore_axis_name="s")
    sc_fn = pl.kernel(
        _sc_body_wrap,
        out_type=jax.ShapeDtypeStruct((CSC, L), jnp.float32),
        mesh=mesh,
        scratch_types=[
            pltpu.VMEM((LW,), jnp.int32),
            pltpu.VMEM((LW,), jnp.int32),
        ] + [
            pltpu.VMEM((ROWS_PER_W, W), jnp.float32)
            for _ in range(NB)
        ] + [
            pltpu.VMEM((CSC, LW), jnp.float32),
            pltpu.SemaphoreType.DMA,
            pltpu.SemaphoreType.DMA,
            pltpu.SemaphoreType.DMA,
        ],
        compiler_params=pltpu.CompilerParams(needs_layout_passes=False),
    )
    tc_out = pl.pallas_call(
        _tc_body,
        grid=(CTC,),
        in_specs=[
            pl.BlockSpec((1, H, W), lambda c: (CSC + c, 0, 0)),
            pl.BlockSpec((4, FH, W), lambda c: (0, 0, 0)),
            pl.BlockSpec((W, FW), lambda c: (0, 0)),
        ],
        out_specs=pl.BlockSpec((1, FH, FW), lambda c: (c, 0, 0)),
        out_shape=jax.ShapeDtypeStruct((CTC, FH, FW), jnp.float32),
    )(xr, u, r)
    sc_out = sc_fn(xr, sidx)

    return sc_out, tc_out


def kernel(x):
    b, c, h, w = x.shape
    sidx = jnp.asarray(_SIDX)
    u = jnp.asarray(_U)
    xr = x.reshape(C, H, W)
    sc_out, tc_out = _run(xr, sidx, u, jnp.asarray(_R))
    out = jnp.concatenate([sc_out, tc_out.reshape(CTC, L)], axis=0)
    return out.reshape(1, C, L)


# speedup vs baseline: 7.0823x; 1.2805x over previous
import jax
import jax.numpy as jnp
import numpy as np
from jax import lax
from jax.experimental import pallas as pl
from jax.experimental.pallas import tpu as pltpu
from jax.experimental.pallas import tpu_sc as plsc

C = 96
H = 512
W = 512
FH = 128
FW = 128
L = FH * FW
NW = 32
PR_PER_W = FH // NW
ROWS_PER_W = 4 * PR_PER_W
LW = PR_PER_W * FW
CG = 4
NB = 2 * CG

with jax.default_device(jax.devices("cpu")[0]):
    _SIDX = np.asarray(
        jax.random.randint(jax.random.key(42), (L,), 0, 16, jnp.int32))

_LL = np.arange(L)
_ROW = 4 * ((_LL // FW) % PR_PER_W) + (_SIDX >> 2)
_COL = 4 * (_LL % FW) + (_SIDX & 3)
_IBUF = ((_ROW << 9) | _COL).astype(np.int32)


def _slab_src(x_hbm, wid, c):
    return x_hbm.at[pl.ds(c * H + wid * ROWS_PER_W, ROWS_PER_W), :]


def _body(x_hbm, ib_hbm, out_hbm, ibuf, xbufs, obuf, load_sem,
          store_sem, sem_s):
    cid = lax.axis_index("c")
    sid = lax.axis_index("s")
    wid = sid * 2 + cid
    base_l = wid * LW

    pltpu.async_copy(ib_hbm.at[pl.ds(base_l, LW)], ibuf, sem_s)
    for b in range(NB):
        pltpu.async_copy(_slab_src(x_hbm, wid, b), xbufs[b], load_sem)
    pltpu.make_async_copy(ib_hbm.at[pl.ds(base_l, LW)], ibuf, sem_s).wait()

    def group(g, carry):
        for half in range(NB // CG):
            c0 = g * NB + half * CG
            for k in range(CG):
                pltpu.make_async_copy(
                    _slab_src(x_hbm, wid, c0 + k), xbufs[half * CG + k],
                    load_sem,
                ).wait()
            @plsc.parallel_loop(0, LW // 16, unroll=4)
            def _sweep(i):
                p = ibuf[pl.ds(i * 16, 16)]
                row = p >> 9
                col = p & (W - 1)
                for k in range(CG):
                    obuf[c0 + k, pl.ds(i * 16, 16)] = plsc.load_gather(
                        xbufs[half * CG + k], [row, col]
                    )

            @pl.when(c0 + NB + CG <= C)
            def _():
                for k in range(CG):
                    pltpu.async_copy(
                        _slab_src(x_hbm, wid, c0 + NB + k),
                        xbufs[half * CG + k], load_sem,
                    )

            pltpu.async_copy(
                obuf.at[pl.ds(c0, CG), :],
                out_hbm.at[pl.ds(c0, CG), pl.ds(base_l, LW)],
                store_sem,
            )

        return carry

    lax.fori_loop(0, C // NB, group, 0)
    for g in range(C // CG):
        pltpu.make_async_copy(
            obuf.at[pl.ds(g * CG, CG), :],
            out_hbm.at[pl.ds(g * CG, CG), pl.ds(base_l, LW)],
            store_sem,
        ).wait()


def _body_wrap(x_hbm, ib_hbm, out_hbm, ibuf, *rest):
    xbufs = rest[:NB]
    obuf, load_sem, store_sem, sem_s = rest[NB:]
    _body(x_hbm, ib_hbm, out_hbm, ibuf, xbufs, obuf, load_sem,
          store_sem, sem_s)


@jax.jit
def _run(xr, sidx):
    mesh = plsc.VectorSubcoreMesh(core_axis_name="c", subcore_axis_name="s")
    kfn = pl.kernel(
        _body_wrap,
        out_type=jax.ShapeDtypeStruct((C, L), jnp.float32),
        mesh=mesh,
        scratch_types=[
            pltpu.VMEM((LW,), jnp.int32),
        ] + [
            pltpu.VMEM((ROWS_PER_W, W), jnp.float32)
            for _ in range(NB)
        ] + [
            pltpu.VMEM((C, LW), jnp.float32),
            pltpu.SemaphoreType.DMA,
            pltpu.SemaphoreType.DMA,
            pltpu.SemaphoreType.DMA,
        ],
        compiler_params=pltpu.CompilerParams(needs_layout_passes=False),
    )
    return kfn(xr, sidx)


def kernel(x):
    b, c, h, w = x.shape
    sidx = jnp.asarray(_IBUF)
    xr = x.reshape(C * H, W)
    out = _run(xr, sidx)
    return out.reshape(1, C, L)
